# baseline (speedup 1.0000x reference)
import jax
import jax.numpy as jnp
from jax import lax
from jax.experimental import pallas as pl
from jax.experimental.pallas import tpu as pltpu
from jax.experimental.pallas import tpu_sc as plsc

NN = 10000
NE = 160000
DD = 128
NB = 2
NTAIL = 34
NREL = 102
EPS = 1e-5

NSUB = 16
EPT = 10080
EP = EPT * NSUB
CH = 56
NCHPT = EPT // CH
NIT = NCHPT // 3
ACC_ROWS = 10000

ROWS = 1000
NBLK = NN // ROWS
GRID = NB * NBLK


def _msg_body(hid, packed, rel, agg,
              acc, rel_sp,
              gA, gB, h0, h1, h2, r0, r1, r2,
              sh0, sh1, sh2, sr0, sr1, sr2, ss0, ss1, ss2, sgA, sgB):
    c = lax.axis_index("c")
    s = lax.axis_index("s")
    z16 = jnp.zeros((16,), jnp.float32)

    def zrow(j, carry):
        for k in range(8):
            h0[j, pl.ds(k * 16, 16)] = z16
        return carry
    lax.fori_loop(0, CH, zrow, 0)

    zb = s * 625
    for k in range(11):
        pltpu.sync_copy(h0, acc.at[pl.ds(zb + k * CH, CH)])
    pltpu.sync_copy(h0.at[pl.ds(0, 9)], acc.at[pl.ds(zb + 616, 9)])

    @pl.when(s == 0)
    def _():
        pltpu.sync_copy(rel.at[pl.ds(0, CH)], h0)
        pltpu.sync_copy(h0, rel_sp.at[pl.ds(0, CH)])
        pltpu.sync_copy(rel.at[pl.ds(CH, 48)], h0.at[pl.ds(0, 48)])
        pltpu.sync_copy(h0.at[pl.ds(0, 48)], rel_sp.at[pl.ds(CH, 48)])

    plsc.subcore_barrier()

    gbase = (c * NSUB + s) * NIT
    HS = (h0, h1, h2)
    SH = (sh0, sh1, sh2)
    SR = (sr0, sr1, sr2)
    SS = (ss0, ss1, ss2)

    def _gather(g, p):
        pltpu.async_copy(hid.at[g.at[4 * p]], HS[p], SH[p])
        pltpu.async_copy(rel_sp.at[g.at[4 * p + 1]], r0 if p == 0 else
                         (r1 if p == 1 else r2), SR[p])

    def _proc(g, p):
        hb = HS[p]
        rb = r0 if p == 0 else (r1 if p == 1 else r2)
        pltpu.make_async_copy(hid.at[g.at[4 * p]], hb, SH[p]).wait()
        pltpu.make_async_copy(rel_sp.at[g.at[4 * p + 1]], rb, SR[p]).wait()

        def mulrow(j, cc):
            j0 = 2 * j
            j1 = 2 * j + 1
            for k in range(8):
                sl = pl.ds(k * 16, 16)
                hb[j0, sl] = hb[j0, sl] * rb[j0, sl]
            for k in range(8):
                sl = pl.ds(k * 16, 16)
                hb[j1, sl] = hb[j1, sl] * rb[j1, sl]
            return cc
        lax.fori_loop(0, CH // 2, mulrow, 0)
        pltpu.async_copy(hb, acc.at[g.at[4 * p + 2]], SS[p], add=True)

    def _drain(g, p):
        pltpu.make_async_copy(HS[p], acc.at[g.at[4 * p + 2]], SS[p]).wait()

    def _gwait(g, sg, i):
        pltpu.make_async_copy(packed.at[i], g, sg).wait()

    pltpu.async_copy(packed.at[gbase], gA, sgA)
    pltpu.async_copy(packed.at[gbase + 1], gB, sgB)
    _gwait(gA, sgA, gbase)
    _gather(gA, 0)
    _gather(gA, 1)
    _gather(gA, 2)
    _gwait(gB, sgB, gbase + 1)

    MIT = NIT // 2

    def outer(m, carry):
        _proc(gA, 0)
        _proc(gA, 1)

        @pl.when(m > 0)
        def _():
            _gwait(gB, sgB, gbase + 2 * m + 1)
        _drain(gA, 0)
        _gather(gB, 0)
        _proc(gA, 2)
        _drain(gA, 1)
        _gather(gB, 1)
        _drain(gA, 2)
        _gather(gB, 2)

        @pl.when(m < MIT - 1)
        def _():
            pltpu.async_copy(packed.at[gbase + 2 * m + 2], gA, sgA)

        _proc(gB, 0)
        _proc(gB, 1)

        @pl.when(m < MIT - 1)
        def _():
            _gwait(gA, sgA, gbase + 2 * m + 2)
            _drain(gB, 0)
            _gather(gA, 0)
        _proc(gB, 2)

        @pl.when(m < MIT - 1)
        def _():
            _drain(gB, 1)
            _gather(gA, 1)
            _drain(gB, 2)
            _gather(gA, 2)
            pltpu.async_copy(packed.at[gbase + 2 * m + 3], gB, sgB)
        return carry
    lax.fori_loop(0, MIT, outer, 0)
    _drain(gB, 0)
    _drain(gB, 1)
    _drain(gB, 2)

    plsc.subcore_barrier()

    ob = s * 632

    @pl.when(s < 15)
    def _():
        for k in range(11):
            pltpu.sync_copy(acc.at[pl.ds(ob + k * CH, CH)], h0)
            pltpu.sync_copy(h0, agg.at[pl.ds(c * NN + ob + k * CH, CH)])
        pltpu.sync_copy(acc.at[pl.ds(ob + 616, 16)], h0.at[pl.ds(0, 16)])
        pltpu.sync_copy(h0.at[pl.ds(0, 16)],
                        agg.at[pl.ds(c * NN + ob + 616, 16)])

    @pl.when(s == 15)
    def _():
        for k in range(9):
            pltpu.sync_copy(acc.at[pl.ds(ob + k * CH, CH)], h0)
            pltpu.sync_copy(h0, agg.at[pl.ds(c * NN + ob + k * CH, CH)])
        pltpu.sync_copy(acc.at[pl.ds(ob + 504, 16)], h0.at[pl.ds(0, 16)])
        pltpu.sync_copy(h0.at[pl.ds(0, 16)],
                        agg.at[pl.ds(c * NN + ob + 504, 16)])


import functools

def _msg1_body(packed, rel, hb16, agg,
               acc, rel_sp,
               gA, gB, r0, r1, r2, hv,
               sr0, sr1, sr2, ss0, ss1, ss2, sgA, sgB):
    c = lax.axis_index("c")
    s = lax.axis_index("s")
    z16 = jnp.zeros((16,), jnp.float32)

    def zrow(j, carry):
        for k in range(8):
            r0[j, pl.ds(k * 16, 16)] = z16
        return carry
    lax.fori_loop(0, CH, zrow, 0)

    zb = s * 626
    for k in range(11):
        pltpu.sync_copy(r0, acc.at[pl.ds(zb + k * CH, CH)])
    pltpu.sync_copy(r0.at[pl.ds(0, 10)], acc.at[pl.ds(zb + 616, 10)])

    @pl.when(s == 0)
    def _():
        pltpu.sync_copy(rel.at[pl.ds(0, CH)], r0)
        pltpu.sync_copy(r0, rel_sp.at[pl.ds(0, CH)])
        pltpu.sync_copy(rel.at[pl.ds(CH, 48)], r0.at[pl.ds(0, 48)])
        pltpu.sync_copy(r0.at[pl.ds(0, 48)], rel_sp.at[pl.ds(CH, 48)])

    pltpu.sync_copy(hb16.at[c], hv)
    plsc.subcore_barrier()

    gbase = (c * NSUB + s) * NIT
    RS = (r0, r1, r2)
    SR = (sr0, sr1, sr2)
    SS = (ss0, ss1, ss2)

    def _gather(g, p):
        pltpu.async_copy(rel_sp.at[g.at[4 * p + 1]], RS[p], SR[p])

    def _proc(g, p):
        rb = RS[p]
        pltpu.make_async_copy(rel_sp.at[g.at[4 * p + 1]], rb, SR[p]).wait()
        hs = hv[...]
        for o in (0, 16, 32, 40):
            sl = pl.ds(o, 16)
            sv = g[4 * p, sl]
            dv = g[4 * p + 2, sl]
            g[4 * p + 2, sl] = jnp.where(sv == hs, dv,
                                         jnp.full((16,), NN, jnp.int32))
        pltpu.async_copy(rb, acc.at[g.at[4 * p + 2]], SS[p], add=True)

    def _drain(g, p):
        pltpu.make_async_copy(RS[p], acc.at[g.at[4 * p + 2]], SS[p]).wait()

    def _gwait(g, sg, i):
        pltpu.make_async_copy(packed.at[i], g, sg).wait()

    pltpu.async_copy(packed.at[gbase], gA, sgA)
    pltpu.async_copy(packed.at[gbase + 1], gB, sgB)
    _gwait(gA, sgA, gbase)
    _gather(gA, 0)
    _gather(gA, 1)
    _gather(gA, 2)
    _gwait(gB, sgB, gbase + 1)

    MIT = NIT // 2

    def outer(m, carry):
        _proc(gA, 0)
        _proc(gA, 1)

        @pl.when(m > 0)
        def _():
            _gwait(gB, sgB, gbase + 2 * m + 1)
        _drain(gA, 0)
        _gather(gB, 0)
        _proc(gA, 2)
        _drain(gA, 1)
        _gather(gB, 1)
        _drain(gA, 2)
        _gather(gB, 2)

        @pl.when(m < MIT - 1)
        def _():
            pltpu.async_copy(packed.at[gbase + 2 * m + 2], gA, sgA)

        _proc(gB, 0)
        _proc(gB, 1)

        @pl.when(m < MIT - 1)
        def _():
            _gwait(gA, sgA, gbase + 2 * m + 2)
            _drain(gB, 0)
            _gather(gA, 0)
        _proc(gB, 2)

        @pl.when(m < MIT - 1)
        def _():
            _drain(gB, 1)
            _gather(gA, 1)
            _drain(gB, 2)
            _gather(gA, 2)
            pltpu.async_copy(packed.at[gbase + 2 * m + 3], gB, sgB)
        return carry
    lax.fori_loop(0, MIT, outer, 0)
    _drain(gB, 0)
    _drain(gB, 1)
    _drain(gB, 2)

    plsc.subcore_barrier()

    ob = s * 632

    @pl.when(s < 15)
    def _():
        for k in range(11):
            pltpu.sync_copy(acc.at[pl.ds(ob + k * CH, CH)], r0)
            pltpu.sync_copy(r0, agg.at[pl.ds(c * NN + ob + k * CH, CH)])
        pltpu.sync_copy(acc.at[pl.ds(ob + 616, 16)], r0.at[pl.ds(0, 16)])
        pltpu.sync_copy(r0.at[pl.ds(0, 16)],
                        agg.at[pl.ds(c * NN + ob + 616, 16)])

    @pl.when(s == 15)
    def _():
        for k in range(9):
            pltpu.sync_copy(acc.at[pl.ds(ob + k * CH, CH)], r0)
            pltpu.sync_copy(r0, agg.at[pl.ds(c * NN + ob + k * CH, CH)])
        pltpu.sync_copy(acc.at[pl.ds(ob + 504, 16)], r0.at[pl.ds(0, 16)])
        pltpu.sync_copy(r0.at[pl.ds(0, 16)],
                        agg.at[pl.ds(c * NN + ob + 504, 16)])


@functools.lru_cache(maxsize=None)
def _get_msg1_kernel():
    return pl.kernel(
        _msg1_body,
        out_type=jax.ShapeDtypeStruct((NB * NN, DD), jnp.float32),
        mesh=plsc.VectorSubcoreMesh(core_axis_name="c", subcore_axis_name="s", num_cores=2, num_subcores=16),
        scratch_types=[
            pltpu.VMEM_SHARED((NN + 16, DD), jnp.float32),
            pltpu.VMEM_SHARED((104, DD), jnp.float32),
            pltpu.VMEM((12, CH), jnp.int32),
            pltpu.VMEM((12, CH), jnp.int32),
            pltpu.VMEM((CH, DD), jnp.float32),
            pltpu.VMEM((CH, DD), jnp.float32),
            pltpu.VMEM((CH, DD), jnp.float32),
            pltpu.VMEM((16,), jnp.int32),
            pltpu.SemaphoreType.DMA,
            pltpu.SemaphoreType.DMA,
            pltpu.SemaphoreType.DMA,
            pltpu.SemaphoreType.DMA,
            pltpu.SemaphoreType.DMA,
            pltpu.SemaphoreType.DMA,
            pltpu.SemaphoreType.DMA,
            pltpu.SemaphoreType.DMA,
        ],
    )


@functools.lru_cache(maxsize=None)
def _get_msg_kernel():
    return pl.kernel(
    _msg_body,
    out_type=jax.ShapeDtypeStruct((NB * NN, DD), jnp.float32),
    mesh=plsc.VectorSubcoreMesh(core_axis_name="c", subcore_axis_name="s", num_cores=2, num_subcores=16),
    scratch_types=[
        pltpu.VMEM_SHARED((ACC_ROWS, DD), jnp.float32),
        pltpu.VMEM_SHARED((104, DD), jnp.float32),
        pltpu.VMEM((12, CH), jnp.int32),
        pltpu.VMEM((12, CH), jnp.int32),
        pltpu.VMEM((CH, DD), jnp.float32),
        pltpu.VMEM((CH, DD), jnp.float32),
        pltpu.VMEM((CH, DD), jnp.float32),
        pltpu.VMEM((CH, DD), jnp.float32),
        pltpu.VMEM((CH, DD), jnp.float32),
        pltpu.VMEM((CH, DD), jnp.float32),
        pltpu.SemaphoreType.DMA,
        pltpu.SemaphoreType.DMA,
        pltpu.SemaphoreType.DMA,
        pltpu.SemaphoreType.DMA,
        pltpu.SemaphoreType.DMA,
        pltpu.SemaphoreType.DMA,
        pltpu.SemaphoreType.DMA,
        pltpu.SemaphoreType.DMA,
        pltpu.SemaphoreType.DMA,
        pltpu.SemaphoreType.DMA,
        pltpu.SemaphoreType.DMA,
    ],
    )


def _dense1_body(hidx_ref, oh_ref, query_ref, agg_ref, w1_ref, w2_ref, bb_ref,
                 g_ref, be_ref, out_ref, qout_ref):
    i = pl.program_id(0)
    b = i // NBLK
    base_n = (i % NBLK) * ROWS
    q = jnp.dot(oh_ref[...], query_ref[...], preferred_element_type=jnp.float32, precision=lax.Precision.HIGHEST)
    qout_ref[...] = q
    hi = jnp.where(b == 0, hidx_ref[0, 0], hidx_ref[0, 1])
    mask = (lax.broadcasted_iota(jnp.int32, (ROWS, 1), 0) + base_n == hi
            ).astype(jnp.float32)
    qsel = jnp.where(b == 0, q[0:1, :], q[1:2, :])
    h = mask * qsel
    a = agg_ref[...] * qsel + h
    out = (jnp.dot(h, w1_ref[...], preferred_element_type=jnp.float32, precision=lax.Precision.HIGHEST)
           + jnp.dot(a, w2_ref[...], preferred_element_type=jnp.float32, precision=lax.Precision.HIGHEST)
           + bb_ref[...])
    mean = jnp.mean(out, axis=-1, keepdims=True)
    var = jnp.mean((out - mean) * (out - mean), axis=-1, keepdims=True)
    out = (out - mean) / jnp.sqrt(var + EPS) * g_ref[...] + be_ref[...]
    out_ref[...] = jnp.maximum(out, 0.0) + h


def _dense1_call(aggr, w1, w2, bb, g, be, hidx, oh, qpad):
    return pl.pallas_call(
        _dense1_body,
        grid=(GRID,),
        in_specs=[
            pl.BlockSpec(memory_space=pltpu.SMEM),
            pl.BlockSpec((8, 104), lambda i: (0, 0)),
            pl.BlockSpec((104, DD), lambda i: (0, 0)),
            pl.BlockSpec((ROWS, DD), lambda i: (i, 0)),
            pl.BlockSpec((DD, DD), lambda i: (0, 0)),
            pl.BlockSpec((DD, DD), lambda i: (0, 0)),
            pl.BlockSpec((1, DD), lambda i: (0, 0)),
            pl.BlockSpec((1, DD), lambda i: (0, 0)),
            pl.BlockSpec((1, DD), lambda i: (0, 0)),
        ],
        out_specs=[
            pl.BlockSpec((ROWS, DD), lambda i: (i, 0)),
            pl.BlockSpec((8, DD), lambda i: (0, 0)),
        ],
        out_shape=[
            jax.ShapeDtypeStruct((NB * NN, DD), jnp.float32),
            jax.ShapeDtypeStruct((8, DD), jnp.float32),
        ],
    )(hidx, oh, qpad, aggr, w1, w2, bb, g, be)


def _dense_body(hidx_ref, hid_ref, agg_ref, w1_ref, w2_ref, bb_ref, g_ref,
                be_ref, q_ref, out_ref):
    i = pl.program_id(0)
    b = i // NBLK
    base_n = (i % NBLK) * ROWS
    h = hid_ref[...]
    a = agg_ref[...]
    hi = jnp.where(b == 0, hidx_ref[0, 0], hidx_ref[0, 1])
    mask = (lax.broadcasted_iota(jnp.int32, (ROWS, 1), 0) + base_n == hi
            ).astype(jnp.float32)
    qsel = jnp.where(b == 0, q_ref[0:1, :], q_ref[1:2, :])
    a = a + mask * qsel
    out = (jnp.dot(h, w1_ref[...], preferred_element_type=jnp.float32, precision=lax.Precision.HIGHEST)
           + jnp.dot(a, w2_ref[...], preferred_element_type=jnp.float32, precision=lax.Precision.HIGHEST)
           + bb_ref[...])
    mean = jnp.mean(out, axis=-1, keepdims=True)
    var = jnp.mean((out - mean) * (out - mean), axis=-1, keepdims=True)
    out = (out - mean) / jnp.sqrt(var + EPS) * g_ref[...] + be_ref[...]
    out_ref[...] = jnp.maximum(out, 0.0) + h


def _dense_call(hid, aggr, w1, w2, bb, g, be, hidx, q):
    return pl.pallas_call(
        _dense_body,
        grid=(GRID,),
        in_specs=[
            pl.BlockSpec(memory_space=pltpu.SMEM),
            pl.BlockSpec((ROWS, DD), lambda i: (i, 0)),
            pl.BlockSpec((ROWS, DD), lambda i: (i, 0)),
            pl.BlockSpec((DD, DD), lambda i: (0, 0)),
            pl.BlockSpec((DD, DD), lambda i: (0, 0)),
            pl.BlockSpec((1, DD), lambda i: (0, 0)),
            pl.BlockSpec((1, DD), lambda i: (0, 0)),
            pl.BlockSpec((1, DD), lambda i: (0, 0)),
            pl.BlockSpec((8, DD), lambda i: (0, 0)),
        ],
        out_specs=pl.BlockSpec((ROWS, DD), lambda i: (i, 0)),
        out_shape=jax.ShapeDtypeStruct((NB * NN, DD), jnp.float32),
    )(hidx, hid, aggr, w1, w2, bb, g, be, q)


def _densco_body(hidx_ref, hid_ref, agg_ref, w1_ref, w2_ref, bb_ref, g_ref,
                 be_ref, q_ref, w1a_ref, w1b_ref, b1_ref, w2p_ref, b2_ref,
                 out_ref):
    i = pl.program_id(0)
    b = i // NBLK
    base_n = (i % NBLK) * ROWS
    h = hid_ref[...]
    a = agg_ref[...]
    hi = jnp.where(b == 0, hidx_ref[0, 0], hidx_ref[0, 1])
    mask = (lax.broadcasted_iota(jnp.int32, (ROWS, 1), 0) + base_n == hi
            ).astype(jnp.float32)
    qsel = jnp.where(b == 0, q_ref[0:1, :], q_ref[1:2, :])
    a = a + mask * qsel
    out = (jnp.dot(h, w1_ref[...], preferred_element_type=jnp.float32, precision=lax.Precision.HIGHEST)
           + jnp.dot(a, w2_ref[...], preferred_element_type=jnp.float32, precision=lax.Precision.HIGHEST)
           + bb_ref[...])
    mean = jnp.mean(out, axis=-1, keepdims=True)
    var = jnp.mean((out - mean) * (out - mean), axis=-1, keepdims=True)
    out = (out - mean) / jnp.sqrt(var + EPS) * g_ref[...] + be_ref[...]
    h2 = jnp.maximum(out, 0.0) + h
    pre = (jnp.dot(qsel, w1b_ref[...], preferred_element_type=jnp.float32, precision=lax.Precision.HIGHEST)
           + b1_ref[...])
    h1 = jnp.maximum(
        jnp.dot(h2, w1a_ref[...], preferred_element_type=jnp.float32, precision=lax.Precision.HIGHEST) + pre, 0.0)
    out_ref[...] = (jnp.dot(h1, w2p_ref[...], preferred_element_type=jnp.float32, precision=lax.Precision.HIGHEST)
                    + b2_ref[...])


def _densco_call(hid, aggr, w1, w2, bb, g, be, hidx, q, w1a, w1b, b1, w2p,
                 b2p):
    return pl.pallas_call(
        _densco_body,
        grid=(GRID,),
        in_specs=[
            pl.BlockSpec(memory_space=pltpu.SMEM),
            pl.BlockSpec((ROWS, DD), lambda i: (i, 0)),
            pl.BlockSpec((ROWS, DD), lambda i: (i, 0)),
            pl.BlockSpec((DD, DD), lambda i: (0, 0)),
            pl.BlockSpec((DD, DD), lambda i: (0, 0)),
            pl.BlockSpec((1, DD), lambda i: (0, 0)),
            pl.BlockSpec((1, DD), lambda i: (0, 0)),
            pl.BlockSpec((1, DD), lambda i: (0, 0)),
            pl.BlockSpec((8, DD), lambda i: (0, 0)),
            pl.BlockSpec((DD, 2 * DD), lambda i: (0, 0)),
            pl.BlockSpec((DD, 2 * DD), lambda i: (0, 0)),
            pl.BlockSpec((1, 2 * DD), lambda i: (0, 0)),
            pl.BlockSpec((2 * DD, DD), lambda i: (0, 0)),
            pl.BlockSpec((1, DD), lambda i: (0, 0)),
        ],
        out_specs=pl.BlockSpec((ROWS, DD), lambda i: (i, 0)),
        out_shape=jax.ShapeDtypeStruct((NB * NN, DD), jnp.float32),
    )(hidx, hid, aggr, w1, w2, bb, g, be, q, w1a, w1b, b1, w2p, b2p)


def _gath_body(sch, ti, out, ti_v, rows, sem):
    c = lax.axis_index("c")
    s = lax.axis_index("s")

    @pl.when((c == 0) & (s == 0))
    def _():
        pltpu.sync_copy(ti, ti_v)
        pltpu.async_copy(sch.at[ti_v], rows, sem).wait()
        pltpu.sync_copy(rows, out)


@functools.lru_cache(maxsize=None)
def _get_gath_kernel():
    return pl.kernel(
        _gath_body,
        out_type=jax.ShapeDtypeStruct((80, DD), jnp.float32),
        mesh=plsc.VectorSubcoreMesh(core_axis_name="c", subcore_axis_name="s", num_cores=2, num_subcores=16),
        scratch_types=[
            pltpu.VMEM((80,), jnp.int32),
            pltpu.VMEM((80, DD), jnp.float32),
            pltpu.SemaphoreType.DMA,
        ],
    )


def kernel(edge_index, edge_type, h_index, t_index, r_index, query, rel_w,
           lin_w, lin_b, ln_scale, ln_bias, mlp_w1, mlp_b1, mlp_w2, mlp_b2):
    f32 = jnp.float32
    i32 = jnp.int32
    src = edge_index[0]
    dstv = edge_index[1]
    padn = EP - NE
    src_p = jnp.concatenate([src, jnp.zeros((padn,), i32)])
    dst_p = jnp.concatenate([dstv, jnp.zeros((padn,), i32)]).reshape(-1, CH)
    et_p = jnp.concatenate([edge_type, jnp.full((padn,), NREL, i32)]
                           ).reshape(-1, CH)
    zc = jnp.zeros_like(et_p)
    packed = jnp.concatenate([
        jnp.stack([(src_p + cc * NN).reshape(-1, CH), et_p, dst_p, zc], axis=1)
        for cc in range(NB)]).reshape(-1, 12, CH)
    rels = [jnp.pad(rel_w[0], ((0, 2), (0, 0))),
            jnp.pad(rel_w[1], ((0, 2), (0, 0)))]
    hidx2 = h_index.reshape(1, NB)
    oh = (jnp.arange(104, dtype=i32)[None, :]
          == jnp.pad(r_index, (0, 6), constant_values=-1)[:, None]).astype(f32)
    qpad = jnp.pad(query, ((0, 2), (0, 0)))

    hglob = h_index.astype(i32) + NN * jnp.arange(NB, dtype=i32)
    hb16 = jnp.broadcast_to(hglob[:, None], (NB, 16))
    aggraw = _get_msg1_kernel()(packed, rels[0], hb16)
    hid, q8 = _dense1_call(aggraw, lin_w[0, :DD, :], lin_w[0, DD:, :],
                           lin_b[0].reshape(1, DD), ln_scale[0].reshape(1, DD),
                           ln_bias[0].reshape(1, DD), hidx2, oh, qpad)

    aggr = _get_msg_kernel()(hid, packed, rels[1])
    w1a = mlp_w1[:DD, :]
    w1b = mlp_w1[DD:, :]
    w2p = jnp.pad(mlp_w2, ((0, 0), (0, DD - 1)))
    b2p = jnp.pad(mlp_b2.reshape(1, 1), ((0, 0), (0, DD - 1)))
    scoretab = _densco_call(hid, aggr, lin_w[1, :DD, :], lin_w[1, DD:, :],
                            lin_b[1].reshape(1, DD), ln_scale[1].reshape(1, DD),
                            ln_bias[1].reshape(1, DD), hidx2, q8,
                            w1a, w1b, mlp_b1.reshape(1, 2 * DD), w2p, b2p)

    tflat = (t_index.astype(i32)
             + NN * jnp.arange(NB, dtype=i32)[:, None]).reshape(-1)
    tflat = jnp.concatenate([tflat, jnp.zeros((80 - NB * NTAIL,), i32)])
    out80 = _get_gath_kernel()(scoretab, tflat)
    return out80[:NB * NTAIL, 0].reshape(NB, NTAIL)


# speedup vs baseline: 1.0017x; 1.0017x over previous
import jax
import jax.numpy as jnp
from jax import lax
from jax.experimental import pallas as pl
from jax.experimental.pallas import tpu as pltpu
from jax.experimental.pallas import tpu_sc as plsc

NN = 10000
NE = 160000
DD = 128
NB = 2
NTAIL = 34
NREL = 102
EPS = 1e-5

NSUB = 16
EPT = 10080
EP = EPT * NSUB
CH = 56
NCHPT = EPT // CH
NIT = NCHPT // 3
ACC_ROWS = 10000

ROWS = 1000
NBLK = NN // ROWS
GRID = NB * NBLK


def _msg_body(hid, packed, rel, agg,
              acc, rel_sp,
              gA, gB, h0, h1, h2, r0, r1, r2,
              sh0, sh1, sh2, sr0, sr1, sr2, ss0, ss1, ss2, sgA, sgB):
    c = lax.axis_index("c")
    s = lax.axis_index("s")
    z16 = jnp.zeros((16,), jnp.float32)

    def zrow(j, carry):
        for k in range(8):
            h0[j, pl.ds(k * 16, 16)] = z16
        return carry
    lax.fori_loop(0, CH, zrow, 0)

    zb = s * 625
    for k in range(11):
        pltpu.sync_copy(h0, acc.at[pl.ds(zb + k * CH, CH)])
    pltpu.sync_copy(h0.at[pl.ds(0, 9)], acc.at[pl.ds(zb + 616, 9)])

    @pl.when(s == 0)
    def _():
        pltpu.sync_copy(rel.at[pl.ds(0, CH)], h0)
        pltpu.sync_copy(h0, rel_sp.at[pl.ds(0, CH)])
        pltpu.sync_copy(rel.at[pl.ds(CH, 48)], h0.at[pl.ds(0, 48)])
        pltpu.sync_copy(h0.at[pl.ds(0, 48)], rel_sp.at[pl.ds(CH, 48)])

    plsc.subcore_barrier()

    gbase = (c * NSUB + s) * NIT
    HS = (h0, h1, h2)
    SH = (sh0, sh1, sh2)
    SR = (sr0, sr1, sr2)
    SS = (ss0, ss1, ss2)

    def _gather(g, p):
        pltpu.async_copy(hid.at[g.at[4 * p]], HS[p], SH[p])
        pltpu.async_copy(rel_sp.at[g.at[4 * p + 1]], r0 if p == 0 else
                         (r1 if p == 1 else r2), SR[p])

    def _proc(g, p):
        hb = HS[p]
        rb = r0 if p == 0 else (r1 if p == 1 else r2)
        pltpu.make_async_copy(hid.at[g.at[4 * p]], hb, SH[p]).wait()
        pltpu.make_async_copy(rel_sp.at[g.at[4 * p + 1]], rb, SR[p]).wait()

        def mulrow(j, cc):
            j0 = 2 * j
            j1 = 2 * j + 1
            for k in range(8):
                sl = pl.ds(k * 16, 16)
                hb[j0, sl] = hb[j0, sl] * rb[j0, sl]
            for k in range(8):
                sl = pl.ds(k * 16, 16)
                hb[j1, sl] = hb[j1, sl] * rb[j1, sl]
            return cc
        lax.fori_loop(0, CH // 2, mulrow, 0)
        pltpu.async_copy(hb, acc.at[g.at[4 * p + 2]], SS[p], add=True)

    def _drain(g, p):
        pltpu.make_async_copy(HS[p], acc.at[g.at[4 * p + 2]], SS[p]).wait()

    def _gwait(g, sg, i):
        pltpu.make_async_copy(packed.at[i], g, sg).wait()

    pltpu.async_copy(packed.at[gbase], gA, sgA)
    pltpu.async_copy(packed.at[gbase + 1], gB, sgB)
    _gwait(gA, sgA, gbase)
    _gather(gA, 0)
    _gather(gA, 1)
    _gather(gA, 2)
    _gwait(gB, sgB, gbase + 1)

    MIT = NIT // 2

    def outer(m, carry):
        _proc(gA, 0)
        _proc(gA, 1)

        @pl.when(m > 0)
        def _():
            _gwait(gB, sgB, gbase + 2 * m + 1)
        _drain(gA, 0)
        _gather(gB, 0)
        _proc(gA, 2)
        _drain(gA, 1)
        _gather(gB, 1)
        _drain(gA, 2)
        _gather(gB, 2)

        @pl.when(m < MIT - 1)
        def _():
            pltpu.async_copy(packed.at[gbase + 2 * m + 2], gA, sgA)

        _proc(gB, 0)
        _proc(gB, 1)

        @pl.when(m < MIT - 1)
        def _():
            _gwait(gA, sgA, gbase + 2 * m + 2)
            _drain(gB, 0)
            _gather(gA, 0)
        _proc(gB, 2)

        @pl.when(m < MIT - 1)
        def _():
            _drain(gB, 1)
            _gather(gA, 1)
            _drain(gB, 2)
            _gather(gA, 2)
            pltpu.async_copy(packed.at[gbase + 2 * m + 3], gB, sgB)
        return carry
    lax.fori_loop(0, MIT, outer, 0)
    _drain(gB, 0)
    _drain(gB, 1)
    _drain(gB, 2)

    plsc.subcore_barrier()

    ob = s * 632

    @pl.when(s < 15)
    def _():
        for k in range(11):
            pltpu.sync_copy(acc.at[pl.ds(ob + k * CH, CH)], h0)
            pltpu.sync_copy(h0, agg.at[pl.ds(c * NN + ob + k * CH, CH)])
        pltpu.sync_copy(acc.at[pl.ds(ob + 616, 16)], h0.at[pl.ds(0, 16)])
        pltpu.sync_copy(h0.at[pl.ds(0, 16)],
                        agg.at[pl.ds(c * NN + ob + 616, 16)])

    @pl.when(s == 15)
    def _():
        for k in range(9):
            pltpu.sync_copy(acc.at[pl.ds(ob + k * CH, CH)], h0)
            pltpu.sync_copy(h0, agg.at[pl.ds(c * NN + ob + k * CH, CH)])
        pltpu.sync_copy(acc.at[pl.ds(ob + 504, 16)], h0.at[pl.ds(0, 16)])
        pltpu.sync_copy(h0.at[pl.ds(0, 16)],
                        agg.at[pl.ds(c * NN + ob + 504, 16)])


import functools

def _msg1_body(packed, rel, hb16, agg,
               acc, rel_sp,
               gA, gB, r0, r1, r2, hv,
               sr0, sr1, sr2, ss0, ss1, ss2, sgA, sgB):
    c = lax.axis_index("c")
    s = lax.axis_index("s")
    z16 = jnp.zeros((16,), jnp.float32)

    def zrow(j, carry):
        for k in range(8):
            r0[j, pl.ds(k * 16, 16)] = z16
        return carry
    lax.fori_loop(0, CH, zrow, 0)

    zb = s * 626
    for k in range(11):
        pltpu.sync_copy(r0, acc.at[pl.ds(zb + k * CH, CH)])
    pltpu.sync_copy(r0.at[pl.ds(0, 10)], acc.at[pl.ds(zb + 616, 10)])

    @pl.when(s == 0)
    def _():
        pltpu.sync_copy(rel.at[pl.ds(0, CH)], r0)
        pltpu.sync_copy(r0, rel_sp.at[pl.ds(0, CH)])
        pltpu.sync_copy(rel.at[pl.ds(CH, 48)], r0.at[pl.ds(0, 48)])
        pltpu.sync_copy(r0.at[pl.ds(0, 48)], rel_sp.at[pl.ds(CH, 48)])

    pltpu.sync_copy(hb16.at[c], hv)
    plsc.subcore_barrier()

    gbase = (c * NSUB + s) * NIT
    RS = (r0, r1, r2)
    SR = (sr0, sr1, sr2)
    SS = (ss0, ss1, ss2)

    def _gather(g, p):
        pltpu.async_copy(rel_sp.at[g.at[4 * p + 1]], RS[p], SR[p])

    def _proc(g, p):
        rb = RS[p]
        pltpu.make_async_copy(rel_sp.at[g.at[4 * p + 1]], rb, SR[p]).wait()
        hs = hv[...]
        for o in (0, 16, 32, 40):
            sl = pl.ds(o, 16)
            sv = g[4 * p, sl]
            dv = g[4 * p + 2, sl]
            g[4 * p + 2, sl] = jnp.where(sv == hs, dv,
                                         jnp.full((16,), NN, jnp.int32))
        pltpu.async_copy(rb, acc.at[g.at[4 * p + 2]], SS[p], add=True)

    def _drain(g, p):
        pltpu.make_async_copy(RS[p], acc.at[g.at[4 * p + 2]], SS[p]).wait()

    def _gwait(g, sg, i):
        pltpu.make_async_copy(packed.at[i], g, sg).wait()

    pltpu.async_copy(packed.at[gbase], gA, sgA)
    pltpu.async_copy(packed.at[gbase + 1], gB, sgB)
    _gwait(gA, sgA, gbase)
    _gather(gA, 0)
    _gather(gA, 1)
    _gather(gA, 2)
    _gwait(gB, sgB, gbase + 1)

    MIT = NIT // 2

    def outer(m, carry):
        _proc(gA, 0)
        _proc(gA, 1)

        @pl.when(m > 0)
        def _():
            _gwait(gB, sgB, gbase + 2 * m + 1)
        _drain(gA, 0)
        _gather(gB, 0)
        _proc(gA, 2)
        _drain(gA, 1)
        _gather(gB, 1)
        _drain(gA, 2)
        _gather(gB, 2)

        @pl.when(m < MIT - 1)
        def _():
            pltpu.async_copy(packed.at[gbase + 2 * m + 2], gA, sgA)

        _proc(gB, 0)
        _proc(gB, 1)

        @pl.when(m < MIT - 1)
        def _():
            _gwait(gA, sgA, gbase + 2 * m + 2)
            _drain(gB, 0)
            _gather(gA, 0)
        _proc(gB, 2)

        @pl.when(m < MIT - 1)
        def _():
            _drain(gB, 1)
            _gather(gA, 1)
            _drain(gB, 2)
            _gather(gA, 2)
            pltpu.async_copy(packed.at[gbase + 2 * m + 3], gB, sgB)
        return carry
    lax.fori_loop(0, MIT, outer, 0)
    _drain(gB, 0)
    _drain(gB, 1)
    _drain(gB, 2)

    plsc.subcore_barrier()

    ob = s * 632

    @pl.when(s < 15)
    def _():
        for k in range(11):
            pltpu.sync_copy(acc.at[pl.ds(ob + k * CH, CH)], r0)
            pltpu.sync_copy(r0, agg.at[pl.ds(c * NN + ob + k * CH, CH)])
        pltpu.sync_copy(acc.at[pl.ds(ob + 616, 16)], r0.at[pl.ds(0, 16)])
        pltpu.sync_copy(r0.at[pl.ds(0, 16)],
                        agg.at[pl.ds(c * NN + ob + 616, 16)])

    @pl.when(s == 15)
    def _():
        for k in range(9):
            pltpu.sync_copy(acc.at[pl.ds(ob + k * CH, CH)], r0)
            pltpu.sync_copy(r0, agg.at[pl.ds(c * NN + ob + k * CH, CH)])
        pltpu.sync_copy(acc.at[pl.ds(ob + 504, 16)], r0.at[pl.ds(0, 16)])
        pltpu.sync_copy(r0.at[pl.ds(0, 16)],
                        agg.at[pl.ds(c * NN + ob + 504, 16)])


@functools.lru_cache(maxsize=None)
def _get_msg1_kernel():
    return pl.kernel(
        _msg1_body,
        out_type=jax.ShapeDtypeStruct((NB * NN, DD), jnp.float32),
        mesh=plsc.VectorSubcoreMesh(core_axis_name="c", subcore_axis_name="s", num_cores=2, num_subcores=16),
        scratch_types=[
            pltpu.VMEM_SHARED((NN + 16, DD), jnp.float32),
            pltpu.VMEM_SHARED((104, DD), jnp.float32),
            pltpu.VMEM((12, CH), jnp.int32),
            pltpu.VMEM((12, CH), jnp.int32),
            pltpu.VMEM((CH, DD), jnp.float32),
            pltpu.VMEM((CH, DD), jnp.float32),
            pltpu.VMEM((CH, DD), jnp.float32),
            pltpu.VMEM((16,), jnp.int32),
            pltpu.SemaphoreType.DMA,
            pltpu.SemaphoreType.DMA,
            pltpu.SemaphoreType.DMA,
            pltpu.SemaphoreType.DMA,
            pltpu.SemaphoreType.DMA,
            pltpu.SemaphoreType.DMA,
            pltpu.SemaphoreType.DMA,
            pltpu.SemaphoreType.DMA,
        ],
    )


@functools.lru_cache(maxsize=None)
def _get_msg_kernel():
    return pl.kernel(
    _msg_body,
    out_type=jax.ShapeDtypeStruct((NB * NN, DD), jnp.float32),
    mesh=plsc.VectorSubcoreMesh(core_axis_name="c", subcore_axis_name="s", num_cores=2, num_subcores=16),
    scratch_types=[
        pltpu.VMEM_SHARED((ACC_ROWS, DD), jnp.float32),
        pltpu.VMEM_SHARED((104, DD), jnp.float32),
        pltpu.VMEM((12, CH), jnp.int32),
        pltpu.VMEM((12, CH), jnp.int32),
        pltpu.VMEM((CH, DD), jnp.float32),
        pltpu.VMEM((CH, DD), jnp.float32),
        pltpu.VMEM((CH, DD), jnp.float32),
        pltpu.VMEM((CH, DD), jnp.float32),
        pltpu.VMEM((CH, DD), jnp.float32),
        pltpu.VMEM((CH, DD), jnp.float32),
        pltpu.SemaphoreType.DMA,
        pltpu.SemaphoreType.DMA,
        pltpu.SemaphoreType.DMA,
        pltpu.SemaphoreType.DMA,
        pltpu.SemaphoreType.DMA,
        pltpu.SemaphoreType.DMA,
        pltpu.SemaphoreType.DMA,
        pltpu.SemaphoreType.DMA,
        pltpu.SemaphoreType.DMA,
        pltpu.SemaphoreType.DMA,
        pltpu.SemaphoreType.DMA,
    ],
    )


def _dense1_body(hidx_ref, oh_ref, query_ref, agg_ref, w1_ref, w2_ref, bb_ref,
                 g_ref, be_ref, out_ref, qout_ref):
    i = pl.program_id(0)
    b = i // NBLK
    base_n = (i % NBLK) * ROWS
    q = jnp.dot(oh_ref[...], query_ref[...], preferred_element_type=jnp.float32, precision=lax.Precision.HIGHEST)
    qout_ref[...] = q
    hi = jnp.where(b == 0, hidx_ref[0, 0], hidx_ref[0, 1])
    mask = (lax.broadcasted_iota(jnp.int32, (ROWS, 1), 0) + base_n == hi
            ).astype(jnp.float32)
    qsel = jnp.where(b == 0, q[0:1, :], q[1:2, :])
    h = mask * qsel
    a = agg_ref[...] * qsel + h
    out = (jnp.dot(h, w1_ref[...], preferred_element_type=jnp.float32, precision=lax.Precision.HIGHEST)
           + jnp.dot(a, w2_ref[...], preferred_element_type=jnp.float32, precision=lax.Precision.HIGHEST)
           + bb_ref[...])
    mean = jnp.mean(out, axis=-1, keepdims=True)
    var = jnp.mean((out - mean) * (out - mean), axis=-1, keepdims=True)
    out = (out - mean) / jnp.sqrt(var + EPS) * g_ref[...] + be_ref[...]
    out_ref[...] = jnp.maximum(out, 0.0) + h


def _dense1_call(aggr, w1, w2, bb, g, be, hidx, oh, qpad):
    return pl.pallas_call(
        _dense1_body,
        grid=(GRID,),
        in_specs=[
            pl.BlockSpec(memory_space=pltpu.SMEM),
            pl.BlockSpec((8, 104), lambda i: (0, 0)),
            pl.BlockSpec((104, DD), lambda i: (0, 0)),
            pl.BlockSpec((ROWS, DD), lambda i: (i, 0)),
            pl.BlockSpec((DD, DD), lambda i: (0, 0)),
            pl.BlockSpec((DD, DD), lambda i: (0, 0)),
            pl.BlockSpec((1, DD), lambda i: (0, 0)),
            pl.BlockSpec((1, DD), lambda i: (0, 0)),
            pl.BlockSpec((1, DD), lambda i: (0, 0)),
        ],
        out_specs=[
            pl.BlockSpec((ROWS, DD), lambda i: (i, 0)),
            pl.BlockSpec((8, DD), lambda i: (0, 0)),
        ],
        out_shape=[
            jax.ShapeDtypeStruct((NB * NN, DD), jnp.float32),
            jax.ShapeDtypeStruct((8, DD), jnp.float32),
        ],
    )(hidx, oh, qpad, aggr, w1, w2, bb, g, be)


def _densco_body(hidx_ref, hid_ref, agg_ref, w1_ref, w2_ref, bb_ref, g_ref,
                 be_ref, q_ref, w1a_ref, w1b_ref, b1_ref, w2p_ref, b2_ref,
                 out_ref):
    i = pl.program_id(0)
    b = i // NBLK
    base_n = (i % NBLK) * ROWS
    h = hid_ref[...]
    a = agg_ref[...]
    hi = jnp.where(b == 0, hidx_ref[0, 0], hidx_ref[0, 1])
    mask = (lax.broadcasted_iota(jnp.int32, (ROWS, 1), 0) + base_n == hi
            ).astype(jnp.float32)
    qsel = jnp.where(b == 0, q_ref[0:1, :], q_ref[1:2, :])
    a = a + mask * qsel
    out = (jnp.dot(h, w1_ref[...], preferred_element_type=jnp.float32, precision=lax.Precision.HIGHEST)
           + jnp.dot(a, w2_ref[...], preferred_element_type=jnp.float32, precision=lax.Precision.HIGHEST)
           + bb_ref[...])
    mean = jnp.mean(out, axis=-1, keepdims=True)
    var = jnp.mean((out - mean) * (out - mean), axis=-1, keepdims=True)
    out = (out - mean) / jnp.sqrt(var + EPS) * g_ref[...] + be_ref[...]
    h2 = jnp.maximum(out, 0.0) + h
    pre = (jnp.dot(qsel, w1b_ref[...], preferred_element_type=jnp.float32, precision=lax.Precision.HIGHEST)
           + b1_ref[...])
    h1 = jnp.maximum(
        jnp.dot(h2, w1a_ref[...], preferred_element_type=jnp.float32, precision=lax.Precision.HIGHEST) + pre, 0.0)
    out_ref[...] = (jnp.dot(h1, w2p_ref[...], preferred_element_type=jnp.float32, precision=lax.Precision.HIGHEST)
                    + b2_ref[...])


def _densco_call(hid, aggr, w1, w2, bb, g, be, hidx, q, w1a, w1b, b1, w2p,
                 b2p):
    return pl.pallas_call(
        _densco_body,
        grid=(GRID,),
        in_specs=[
            pl.BlockSpec(memory_space=pltpu.SMEM),
            pl.BlockSpec((ROWS, DD), lambda i: (i, 0)),
            pl.BlockSpec((ROWS, DD), lambda i: (i, 0)),
            pl.BlockSpec((DD, DD), lambda i: (0, 0)),
            pl.BlockSpec((DD, DD), lambda i: (0, 0)),
            pl.BlockSpec((1, DD), lambda i: (0, 0)),
            pl.BlockSpec((1, DD), lambda i: (0, 0)),
            pl.BlockSpec((1, DD), lambda i: (0, 0)),
            pl.BlockSpec((8, DD), lambda i: (0, 0)),
            pl.BlockSpec((DD, 2 * DD), lambda i: (0, 0)),
            pl.BlockSpec((DD, 2 * DD), lambda i: (0, 0)),
            pl.BlockSpec((1, 2 * DD), lambda i: (0, 0)),
            pl.BlockSpec((2 * DD, DD), lambda i: (0, 0)),
            pl.BlockSpec((1, DD), lambda i: (0, 0)),
        ],
        out_specs=pl.BlockSpec((ROWS, DD), lambda i: (i, 0)),
        out_shape=jax.ShapeDtypeStruct((NB * NN, DD), jnp.float32),
    )(hidx, hid, aggr, w1, w2, bb, g, be, q, w1a, w1b, b1, w2p, b2p)


def _gath_body(sch, ti, out, ti_v, rows, sem):
    c = lax.axis_index("c")
    s = lax.axis_index("s")

    @pl.when((c == 0) & (s == 0))
    def _():
        pltpu.sync_copy(ti, ti_v)
        pltpu.async_copy(sch.at[ti_v], rows, sem).wait()
        pltpu.sync_copy(rows, out)


@functools.lru_cache(maxsize=None)
def _get_gath_kernel():
    return pl.kernel(
        _gath_body,
        out_type=jax.ShapeDtypeStruct((80, DD), jnp.float32),
        mesh=plsc.VectorSubcoreMesh(core_axis_name="c", subcore_axis_name="s", num_cores=2, num_subcores=16),
        scratch_types=[
            pltpu.VMEM((80,), jnp.int32),
            pltpu.VMEM((80, DD), jnp.float32),
            pltpu.SemaphoreType.DMA,
        ],
    )


def kernel(edge_index, edge_type, h_index, t_index, r_index, query, rel_w,
           lin_w, lin_b, ln_scale, ln_bias, mlp_w1, mlp_b1, mlp_w2, mlp_b2):
    f32 = jnp.float32
    i32 = jnp.int32
    src = edge_index[0]
    dstv = edge_index[1]
    padn = EP - NE
    src_p = jnp.concatenate([src, jnp.zeros((padn,), i32)])
    dst_p = jnp.concatenate([dstv, jnp.zeros((padn,), i32)]).reshape(-1, CH)
    et_p = jnp.concatenate([edge_type, jnp.full((padn,), NREL, i32)]
                           ).reshape(-1, CH)
    zc = jnp.zeros_like(et_p)
    packed = jnp.concatenate([
        jnp.stack([(src_p + cc * NN).reshape(-1, CH), et_p, dst_p, zc], axis=1)
        for cc in range(NB)]).reshape(-1, 12, CH)
    rels = [jnp.pad(rel_w[0], ((0, 2), (0, 0))),
            jnp.pad(rel_w[1], ((0, 2), (0, 0)))]
    hidx2 = h_index.reshape(1, NB)
    oh = (jnp.arange(104, dtype=i32)[None, :]
          == jnp.pad(r_index, (0, 6), constant_values=-1)[:, None]).astype(f32)
    qpad = jnp.pad(query, ((0, 2), (0, 0)))

    hglob = h_index.astype(i32) + NN * jnp.arange(NB, dtype=i32)
    hb16 = jnp.broadcast_to(hglob[:, None], (NB, 16))
    aggraw = _get_msg1_kernel()(packed, rels[0], hb16)
    hid, q8 = _dense1_call(aggraw, lin_w[0, :DD, :], lin_w[0, DD:, :],
                           lin_b[0].reshape(1, DD), ln_scale[0].reshape(1, DD),
                           ln_bias[0].reshape(1, DD), hidx2, oh, qpad)

    aggr = _get_msg_kernel()(hid, packed, rels[1])
    w1a = mlp_w1[:DD, :]
    w1b = mlp_w1[DD:, :]
    w2p = jnp.pad(mlp_w2, ((0, 0), (0, DD - 1)))
    b2p = jnp.pad(mlp_b2.reshape(1, 1), ((0, 0), (0, DD - 1)))
    scoretab = _densco_call(hid, aggr, lin_w[1, :DD, :], lin_w[1, DD:, :],
                            lin_b[1].reshape(1, DD), ln_scale[1].reshape(1, DD),
                            ln_bias[1].reshape(1, DD), hidx2, q8,
                            w1a, w1b, mlp_b1.reshape(1, 2 * DD), w2p, b2p)

    tflat = (t_index.astype(i32)
             + NN * jnp.arange(NB, dtype=i32)[:, None]).reshape(-1)
    tflat = jnp.concatenate([tflat, jnp.zeros((80 - NB * NTAIL,), i32)])
    out80 = _get_gath_kernel()(scoretab, tflat)
    return out80[:NB * NTAIL, 0].reshape(NB, NTAIL)
